# baseline (speedup 1.0000x reference)
import jax
import jax.numpy as jnp
from jax.experimental import pallas as pl
from jax.experimental.pallas import tpu as pltpu

_SHAPE = (2, 16384, 1024)
_TILE_ROWS = 8


def _copy_update_kernel(key_ref, in_ref, out_ref, tile_ref, sem_big, sem_small):
    big0 = pltpu.make_async_copy(in_ref.at[0], out_ref.at[0], sem_big.at[0])
    big1 = pltpu.make_async_copy(
        in_ref.at[1, pl.ds(_TILE_ROWS, _SHAPE[1] - _TILE_ROWS), :],
        out_ref.at[1, pl.ds(_TILE_ROWS, _SHAPE[1] - _TILE_ROWS), :],
        sem_big.at[1],
    )
    big0.start()
    big1.start()

    small_in = pltpu.make_async_copy(
        in_ref.at[1, pl.ds(0, _TILE_ROWS), :], tile_ref, sem_small
    )
    small_in.start()
    small_in.wait()
    row = jax.lax.broadcasted_iota(jnp.int32, (_TILE_ROWS, _SHAPE[2]), 0)
    col = jax.lax.broadcasted_iota(jnp.int32, (_TILE_ROWS, _SHAPE[2]), 1)
    mask = (row == 0) & (col == 1)
    tile_ref[...] += jnp.where(mask, 2.0 * key_ref[0], 0.0)
    small_out = pltpu.make_async_copy(
        tile_ref, out_ref.at[1, pl.ds(0, _TILE_ROWS), :], sem_small
    )
    small_out.start()
    small_out.wait()
    big0.wait()
    big1.wait()


def kernel(key, cache_next):
    out = pl.pallas_call(
        _copy_update_kernel,
        out_shape=jax.ShapeDtypeStruct(_SHAPE, jnp.float32),
        in_specs=[
            pl.BlockSpec(memory_space=pltpu.SMEM),
            pl.BlockSpec(memory_space=pl.ANY),
        ],
        out_specs=pl.BlockSpec(memory_space=pl.ANY),
        scratch_shapes=[
            pltpu.VMEM((_TILE_ROWS, _SHAPE[2]), jnp.float32),
            pltpu.SemaphoreType.DMA((2,)),
            pltpu.SemaphoreType.DMA,
        ],
    )(key, cache_next)
    return key, out


# speedup vs baseline: 46.9382x; 46.9382x over previous
import jax
import jax.numpy as jnp
from jax.experimental import pallas as pl
from jax.experimental.pallas import tpu as pltpu

_SHAPE = (2, 16384, 1024)
_BLOCK_ROWS = 512


def _copy_update_kernel(key_ref, in_ref, out_ref):
    out_ref[...] = in_ref[...]

    @pl.when(pl.program_id(0) == 0)
    def _():
        row = jax.lax.broadcasted_iota(jnp.int32, (8, 128), 0)
        col = jax.lax.broadcasted_iota(jnp.int32, (8, 128), 1)
        mask = (row == 0) & (col == 1)
        out_ref[1, 0:8, 0:128] = in_ref[1, 0:8, 0:128] + jnp.where(
            mask, 2.0 * key_ref[0], 0.0
        )


def kernel(key, cache_next):
    grid = (_SHAPE[1] // _BLOCK_ROWS,)
    block = (2, _BLOCK_ROWS, _SHAPE[2])
    out = pl.pallas_call(
        _copy_update_kernel,
        grid=grid,
        out_shape=jax.ShapeDtypeStruct(_SHAPE, jnp.float32),
        in_specs=[
            pl.BlockSpec(memory_space=pltpu.SMEM),
            pl.BlockSpec(block, lambda i: (0, i, 0)),
        ],
        out_specs=pl.BlockSpec(block, lambda i: (0, i, 0)),
    )(key, cache_next)
    return key, out
